# baseline (speedup 1.0000x reference)
import functools

import jax
import jax.numpy as jnp
from jax import lax
from jax.experimental import pallas as pl
from jax.experimental.pallas import tpu as pltpu
from jax.experimental.pallas import tpu_sc as plsc

HIDDEN = 768
N_EXPERTS = 8
EPAD = 16
BT = 256
RT = 1024
NW = 32
CH = 128


def _router_body(x_ref, wr_ref, idx_ref, rank_ref, cnt_ref, offs_ref, carry_ref):
    @pl.when(pl.program_id(0) == 0)
    def _init():
        carry_ref[...] = jnp.zeros_like(carry_ref)

    x = x_ref[...]
    logits = lax.dot_general(
        x, wr_ref[...], (((1,), (1,)), ((), ())),
        preferred_element_type=jnp.float32)
    mx = jnp.max(logits, axis=1, keepdims=True)
    eids = lax.broadcasted_iota(jnp.int32, logits.shape, 1)
    idx = jnp.min(jnp.where(logits == mx, eids, N_EXPERTS), axis=1)

    oh = (idx[:, None] == lax.broadcasted_iota(
        jnp.int32, (BT, EPAD), 1)).astype(jnp.float32)
    r = lax.broadcasted_iota(jnp.int32, (BT, BT), 0)
    c = lax.broadcasted_iota(jnp.int32, (BT, BT), 1)
    tril = (c < r).astype(jnp.float32)
    excl = lax.dot_general(
        tril, oh, (((1,), (0,)), ((), ())),
        preferred_element_type=jnp.float32)
    carry = carry_ref[...]
    rank = jnp.sum((excl.astype(jnp.int32) + carry) *
                   oh.astype(jnp.int32), axis=1)
    tot = jnp.sum(oh.astype(jnp.int32), axis=0)
    newc = carry + tot[None, :]
    carry_ref[...] = newc
    idx_ref[...] = idx[:, None]
    rank_ref[...] = rank[:, None]
    cnt_ref[...] = jnp.broadcast_to(newc, (8, EPAD))
    ra = lax.broadcasted_iota(jnp.int32, (EPAD, EPAD), 0)
    ca = lax.broadcasted_iota(jnp.int32, (EPAD, EPAD), 1)
    sup = (ra < ca).astype(jnp.float32)
    chi = (newc >> 8).astype(jnp.float32)
    clo = (newc & 255).astype(jnp.float32)
    ohi = lax.dot_general(chi, sup, (((1,), (0,)), ((), ())),
                          preferred_element_type=jnp.float32)
    olo = lax.dot_general(clo, sup, (((1,), (0,)), ((), ())),
                          preferred_element_type=jnp.float32)
    offs = ohi.astype(jnp.int32) * 256 + olo.astype(jnp.int32)
    offs_ref[...] = jnp.broadcast_to(offs, (8, EPAD))


def _router(x, Wr):
    T, H = x.shape
    return pl.pallas_call(
        _router_body,
        grid=(T // BT,),
        in_specs=[
            pl.BlockSpec((BT, H), lambda i: (i, 0)),
            pl.BlockSpec((N_EXPERTS, H), lambda i: (0, 0)),
        ],
        out_specs=[
            pl.BlockSpec((BT, 1), lambda i: (i, 0)),
            pl.BlockSpec((BT, 1), lambda i: (i, 0)),
            pl.BlockSpec((8, EPAD), lambda i: (0, 0)),
            pl.BlockSpec((8, EPAD), lambda i: (0, 0)),
        ],
        out_shape=[
            jax.ShapeDtypeStruct((T, 1), jnp.int32),
            jax.ShapeDtypeStruct((T, 1), jnp.int32),
            jax.ShapeDtypeStruct((8, EPAD), jnp.int32),
            jax.ShapeDtypeStruct((8, EPAD), jnp.int32),
        ],
        scratch_shapes=[pltpu.VMEM((1, EPAD), jnp.int32)],
    )(x, Wr)


BP = 4096


def _pos_body(idx_ref, rank_ref, offs_ref, pos_ref):
    idxv = idx_ref[...]
    oh = (idxv == lax.broadcasted_iota(
        jnp.int32, (BP, EPAD), 1)).astype(jnp.int32)
    base = jnp.sum(oh * offs_ref[0][None, :], axis=1, keepdims=True)
    pos_ref[...] = base + rank_ref[...]


def _pos(idx2, rank2, offs2):
    T = idx2.shape[0]
    return pl.pallas_call(
        _pos_body,
        grid=(T // BP,),
        in_specs=[
            pl.BlockSpec((BP, 1), lambda i: (i, 0)),
            pl.BlockSpec((BP, 1), lambda i: (i, 0)),
            pl.BlockSpec((8, EPAD), lambda i: (0, 0)),
        ],
        out_specs=pl.BlockSpec((BP, 1), lambda i: (i, 0)),
        out_shape=jax.ShapeDtypeStruct((T, 1), jnp.int32),
    )(idx2, rank2, offs2)


def _make_dispatch(T, H):
    per_w = T // NW
    n_ch = per_w // CH
    mesh = plsc.VectorSubcoreMesh(core_axis_name="c", subcore_axis_name="s")

    @functools.partial(
        pl.kernel, mesh=mesh,
        out_type=jax.ShapeDtypeStruct((T, H), jnp.float32),
        scratch_types=[
            pltpu.VMEM((CH,), jnp.int32),
            pltpu.VMEM((CH, H), jnp.float32),
            pltpu.SemaphoreType.DMA,
        ],
    )
    def dispatch(x_hbm, pos_hbm, xs_hbm, pos_v, rows_v, sem):
        wid = lax.axis_index("s") * 2 + lax.axis_index("c")
        for i in range(n_ch):
            base = wid * per_w + i * CH
            pltpu.sync_copy(pos_hbm.at[pl.ds(base, CH)], pos_v)
            pltpu.sync_copy(x_hbm.at[pl.ds(base, CH)], rows_v)
            pltpu.async_copy(rows_v, xs_hbm.at[pos_v], sem).wait()

    return dispatch


def _grouped_body(cnt_ref, xs_ref, we_ref, out_ref):
    i = pl.program_id(0)
    ts = i * RT
    rid = lax.broadcasted_iota(jnp.int32, (RT, 1), 0) + ts
    lo = 0
    for e in range(N_EXPERTS):
        hi = lo + cnt_ref[e]
        pred = jnp.logical_and(lo < ts + RT, hi > ts)
        first = lo <= ts
        covered = jnp.logical_and(first, hi >= ts + RT)

        def _full(e=e):
            out_ref[...] = lax.dot_general(
                xs_ref[...], we_ref[e], (((1,), (1,)), ((), ())),
                preferred_element_type=jnp.float32)

        def _partial_w(lo=lo, hi=hi, e=e):
            ye = lax.dot_general(
                xs_ref[...], we_ref[e], (((1,), (1,)), ((), ())),
                preferred_element_type=jnp.float32)
            m = (jnp.logical_and(rid >= lo, rid < hi)).astype(jnp.float32)
            out_ref[...] = m * ye

        def _partial_a(lo=lo, hi=hi, e=e):
            ye = lax.dot_general(
                xs_ref[...], we_ref[e], (((1,), (1,)), ((), ())),
                preferred_element_type=jnp.float32)
            m = (jnp.logical_and(rid >= lo, rid < hi)).astype(jnp.float32)
            out_ref[...] += m * ye

        partial = jnp.logical_and(pred, jnp.logical_not(covered))
        pl.when(covered)(_full)
        pl.when(jnp.logical_and(partial, first))(_partial_w)
        pl.when(jnp.logical_and(partial, jnp.logical_not(first)))(_partial_a)
        lo = hi


def _grouped(counts, xs, We):
    T, H = xs.shape
    E = We.shape[0]
    grid_spec = pltpu.PrefetchScalarGridSpec(
        num_scalar_prefetch=1,
        grid=(T // RT,),
        in_specs=[
            pl.BlockSpec((RT, H), lambda i, cnt: (i, 0)),
            pl.BlockSpec((E, H, H), lambda i, cnt: (0, 0, 0)),
        ],
        out_specs=pl.BlockSpec((RT, H), lambda i, cnt: (i, 0)),
    )
    return pl.pallas_call(
        _grouped_body,
        grid_spec=grid_spec,
        out_shape=jax.ShapeDtypeStruct((T, H), jnp.float32),
    )(counts, xs, We)


def _make_combine(T, H):
    per_w = T // NW
    n_ch = per_w // CH
    mesh = plsc.VectorSubcoreMesh(core_axis_name="c", subcore_axis_name="s")

    @functools.partial(
        pl.kernel, mesh=mesh,
        out_type=jax.ShapeDtypeStruct((T, H), jnp.float32),
        scratch_types=[
            pltpu.VMEM((CH,), jnp.int32),
            pltpu.VMEM((CH, H), jnp.float32),
            pltpu.SemaphoreType.DMA,
        ],
    )
    def combine(os_hbm, pos_hbm, out_hbm, pos_v, rows_v, sem):
        wid = lax.axis_index("s") * 2 + lax.axis_index("c")
        for i in range(n_ch):
            base = wid * per_w + i * CH
            pltpu.sync_copy(pos_hbm.at[pl.ds(base, CH)], pos_v)
            pltpu.async_copy(os_hbm.at[pos_v], rows_v, sem).wait()
            pltpu.sync_copy(rows_v, out_hbm.at[pl.ds(base, CH)])

    return combine


@jax.jit
def kernel(x, Wr, We):
    T, H = x.shape
    idx2, rank2, counts2, offs2 = _router(x, Wr)
    counts = counts2[0]
    pos = _pos(idx2, rank2, offs2).reshape(T)
    xs = _make_dispatch(T, H)(x, pos)
    os = _grouped(counts, xs, We)
    return _make_combine(T, H)(os, pos)


# speedup vs baseline: 1.1192x; 1.1192x over previous
import functools

import jax
import jax.numpy as jnp
from jax import lax
from jax.experimental import pallas as pl
from jax.experimental.pallas import tpu as pltpu
from jax.experimental.pallas import tpu_sc as plsc

HIDDEN = 768
N_EXPERTS = 8
EPAD = 16
BT = 512
RT = 1024
NW = 32
CH = 128


def _router_body(x_ref, wr_ref, idx_ref, rank_ref, cnt_ref, offs_ref, carry_ref):
    @pl.when(pl.program_id(0) == 0)
    def _init():
        carry_ref[...] = jnp.zeros_like(carry_ref)

    x = x_ref[...]
    logits = lax.dot_general(
        x, wr_ref[...], (((1,), (1,)), ((), ())),
        preferred_element_type=jnp.float32)
    mx = jnp.max(logits, axis=1, keepdims=True)
    eids = lax.broadcasted_iota(jnp.int32, logits.shape, 1)
    idx = jnp.min(jnp.where(logits == mx, eids, N_EXPERTS), axis=1)

    oh = (idx[:, None] == lax.broadcasted_iota(
        jnp.int32, (BT, EPAD), 1)).astype(jnp.float32)
    r = lax.broadcasted_iota(jnp.int32, (BT, BT), 0)
    c = lax.broadcasted_iota(jnp.int32, (BT, BT), 1)
    tril = (c < r).astype(jnp.float32)
    excl = lax.dot_general(
        tril, oh, (((1,), (0,)), ((), ())),
        preferred_element_type=jnp.float32)
    carry = carry_ref[...]
    rank = jnp.sum((excl.astype(jnp.int32) + carry) *
                   oh.astype(jnp.int32), axis=1)
    tot = jnp.sum(oh.astype(jnp.int32), axis=0)
    newc = carry + tot[None, :]
    carry_ref[...] = newc
    idx_ref[...] = idx[:, None]
    rank_ref[...] = rank[:, None]
    cnt_ref[...] = jnp.broadcast_to(newc, (8, EPAD))
    ra = lax.broadcasted_iota(jnp.int32, (EPAD, EPAD), 0)
    ca = lax.broadcasted_iota(jnp.int32, (EPAD, EPAD), 1)
    sup = (ra < ca).astype(jnp.float32)
    chi = (newc >> 8).astype(jnp.float32)
    clo = (newc & 255).astype(jnp.float32)
    ohi = lax.dot_general(chi, sup, (((1,), (0,)), ((), ())),
                          preferred_element_type=jnp.float32)
    olo = lax.dot_general(clo, sup, (((1,), (0,)), ((), ())),
                          preferred_element_type=jnp.float32)
    offs = ohi.astype(jnp.int32) * 256 + olo.astype(jnp.int32)
    offs_ref[...] = jnp.broadcast_to(offs, (8, EPAD))


def _router(x, Wr):
    T, H = x.shape
    return pl.pallas_call(
        _router_body,
        grid=(T // BT,),
        in_specs=[
            pl.BlockSpec((BT, H), lambda i: (i, 0)),
            pl.BlockSpec((N_EXPERTS, H), lambda i: (0, 0)),
        ],
        out_specs=[
            pl.BlockSpec((BT, 1), lambda i: (i, 0)),
            pl.BlockSpec((BT, 1), lambda i: (i, 0)),
            pl.BlockSpec((8, EPAD), lambda i: (0, 0)),
            pl.BlockSpec((8, EPAD), lambda i: (0, 0)),
        ],
        out_shape=[
            jax.ShapeDtypeStruct((T, 1), jnp.int32),
            jax.ShapeDtypeStruct((T, 1), jnp.int32),
            jax.ShapeDtypeStruct((8, EPAD), jnp.int32),
            jax.ShapeDtypeStruct((8, EPAD), jnp.int32),
        ],
        scratch_shapes=[pltpu.VMEM((1, EPAD), jnp.int32)],
    )(x, Wr)


BP = 4096


def _pos_body(idx_ref, rank_ref, offs_ref, pos_ref):
    idxv = idx_ref[...]
    oh = (idxv == lax.broadcasted_iota(
        jnp.int32, (BP, EPAD), 1)).astype(jnp.int32)
    base = jnp.sum(oh * offs_ref[0][None, :], axis=1, keepdims=True)
    pos_ref[...] = base + rank_ref[...]


def _pos(idx2, rank2, offs2):
    T = idx2.shape[0]
    return pl.pallas_call(
        _pos_body,
        grid=(T // BP,),
        in_specs=[
            pl.BlockSpec((BP, 1), lambda i: (i, 0)),
            pl.BlockSpec((BP, 1), lambda i: (i, 0)),
            pl.BlockSpec((8, EPAD), lambda i: (0, 0)),
        ],
        out_specs=pl.BlockSpec((BP, 1), lambda i: (i, 0)),
        out_shape=jax.ShapeDtypeStruct((T, 1), jnp.int32),
    )(idx2, rank2, offs2)


def _make_dispatch(T, H):
    per_w = T // NW
    n_ch = per_w // CH
    mesh = plsc.VectorSubcoreMesh(core_axis_name="c", subcore_axis_name="s")

    @functools.partial(
        pl.kernel, mesh=mesh,
        out_type=jax.ShapeDtypeStruct((T, H), jnp.float32),
        scratch_types=[
            pltpu.VMEM((CH,), jnp.int32),
            pltpu.VMEM((CH, H), jnp.float32),
            pltpu.SemaphoreType.DMA,
        ],
    )
    def dispatch(x_hbm, pos_hbm, xs_hbm, pos_v, rows_v, sem):
        wid = lax.axis_index("s") * 2 + lax.axis_index("c")
        for i in range(n_ch):
            base = wid * per_w + i * CH
            pltpu.sync_copy(pos_hbm.at[pl.ds(base, CH)], pos_v)
            pltpu.sync_copy(x_hbm.at[pl.ds(base, CH)], rows_v)
            pltpu.async_copy(rows_v, xs_hbm.at[pos_v], sem).wait()

    return dispatch


def _grouped_body(cnt_ref, xs_ref, we_ref, out_ref):
    i = pl.program_id(0)
    ts = i * RT
    rid = lax.broadcasted_iota(jnp.int32, (RT, 1), 0) + ts
    lo = 0
    for e in range(N_EXPERTS):
        hi = lo + cnt_ref[e]
        pred = jnp.logical_and(lo < ts + RT, hi > ts)
        first = lo <= ts
        covered = jnp.logical_and(first, hi >= ts + RT)

        def _full(e=e):
            out_ref[...] = lax.dot_general(
                xs_ref[...], we_ref[e], (((1,), (1,)), ((), ())),
                preferred_element_type=jnp.float32)

        def _partial_w(lo=lo, hi=hi, e=e):
            ye = lax.dot_general(
                xs_ref[...], we_ref[e], (((1,), (1,)), ((), ())),
                preferred_element_type=jnp.float32)
            m = (jnp.logical_and(rid >= lo, rid < hi)).astype(jnp.float32)
            out_ref[...] = m * ye

        def _partial_a(lo=lo, hi=hi, e=e):
            ye = lax.dot_general(
                xs_ref[...], we_ref[e], (((1,), (1,)), ((), ())),
                preferred_element_type=jnp.float32)
            m = (jnp.logical_and(rid >= lo, rid < hi)).astype(jnp.float32)
            out_ref[...] += m * ye

        partial = jnp.logical_and(pred, jnp.logical_not(covered))
        pl.when(covered)(_full)
        pl.when(jnp.logical_and(partial, first))(_partial_w)
        pl.when(jnp.logical_and(partial, jnp.logical_not(first)))(_partial_a)
        lo = hi


def _grouped(counts, xs, We):
    T, H = xs.shape
    E = We.shape[0]
    grid_spec = pltpu.PrefetchScalarGridSpec(
        num_scalar_prefetch=1,
        grid=(T // RT,),
        in_specs=[
            pl.BlockSpec((RT, H), lambda i, cnt: (i, 0)),
            pl.BlockSpec((E, H, H), lambda i, cnt: (0, 0, 0)),
        ],
        out_specs=pl.BlockSpec((RT, H), lambda i, cnt: (i, 0)),
    )
    return pl.pallas_call(
        _grouped_body,
        grid_spec=grid_spec,
        out_shape=jax.ShapeDtypeStruct((T, H), jnp.float32),
    )(counts, xs, We)


def _make_combine(T, H):
    per_w = T // NW
    n_ch = per_w // CH
    mesh = plsc.VectorSubcoreMesh(core_axis_name="c", subcore_axis_name="s")

    @functools.partial(
        pl.kernel, mesh=mesh,
        out_type=jax.ShapeDtypeStruct((T, H), jnp.float32),
        scratch_types=[
            pltpu.VMEM((CH,), jnp.int32),
            pltpu.VMEM((CH, H), jnp.float32),
            pltpu.SemaphoreType.DMA,
        ],
    )
    def combine(os_hbm, pos_hbm, out_hbm, pos_v, rows_v, sem):
        wid = lax.axis_index("s") * 2 + lax.axis_index("c")
        for i in range(n_ch):
            base = wid * per_w + i * CH
            pltpu.sync_copy(pos_hbm.at[pl.ds(base, CH)], pos_v)
            pltpu.async_copy(os_hbm.at[pos_v], rows_v, sem).wait()
            pltpu.sync_copy(rows_v, out_hbm.at[pl.ds(base, CH)])

    return combine


@jax.jit
def kernel(x, Wr, We):
    T, H = x.shape
    idx2, rank2, counts2, offs2 = _router(x, Wr)
    counts = counts2[0]
    pos = _pos(idx2, rank2, offs2).reshape(T)
    xs = _make_dispatch(T, H)(x, pos)
    os = _grouped(counts, xs, We)
    return _make_combine(T, H)(os, pos)
